# baseline (speedup 1.0000x reference)
import functools

import jax
import jax.numpy as jnp
from jax import lax
from jax.experimental import pallas as pl
from jax.experimental.pallas import tpu as pltpu
from jax.experimental.pallas import tpu_sc as plsc


_L = 16


@functools.lru_cache(maxsize=None)
def _make_sc_mean(B: int, S: int, N: int, D: int):
    info = plsc.get_sparse_core_info()
    NC, NS = info.num_cores, info.num_subcores
    NW = NC * NS

    NB = 256 // S
    SPB = NB * S // 128
    assert 128 % S == 0 and D % (2 * _L) == 0
    DW = D // 2
    NG = D // (2 * _L)
    b_per_w = -(-B // NW)
    b_per_w = -(-b_per_w // (2 * NB)) * (2 * NB)
    max_blocks = b_per_w // NB
    tail_rows = B - (B // b_per_w) * b_per_w if B % b_per_w else b_per_w
    chunk_rows = 1
    for c in range(min(tail_rows, b_per_w), 0, -1):
        if tail_rows % c == 0 and b_per_w % c == 0:
            chunk_rows = c
            break
    n_chunks = b_per_w // chunk_rows
    for w in range(NW):
        cnt = min(max_blocks, (B - w * b_per_w) // NB)
        assert cnt >= 2 and cnt % 2 == 0, (w, cnt)
    assert S % 2 == 0
    scale = 1.0 / float(S)

    mesh = plsc.VectorSubcoreMesh(core_axis_name="c", subcore_axis_name="s")

    @functools.partial(
        pl.kernel,
        out_type=jax.ShapeDtypeStruct((B, D), jnp.float32),
        mesh=mesh,
        compiler_params=pltpu.CompilerParams(
            use_tc_tiling_on_sc=False, needs_layout_passes=False),
        scratch_types=[
            pltpu.VMEM((max_blocks * SPB, 128), jnp.int32),
            pltpu.VMEM((NB * S, DW), jnp.int32),
            pltpu.VMEM((NB * S, DW), jnp.int32),
            pltpu.VMEM((b_per_w, D), jnp.float32),
            pltpu.SemaphoreType.DMA,
            pltpu.SemaphoreType.DMA,
        ],
    )
    def sc_mean(idx_hbm, feat_hbm, out_hbm, idx_v, rows0, rows1,
                out_v, sem0, sem1):
        wid = lax.axis_index("s") * NC + lax.axis_index("c")
        base_row = wid * b_per_w
        nblocks = jnp.minimum(max_blocks, (B - base_row) // NB)

        pltpu.sync_copy(idx_hbm.at[wid], idx_v)

        def start_gather(block, rows_v, sem):
            for j in range(SPB):
                pltpu.make_async_copy(
                    feat_hbm.at[idx_v.at[block * SPB + j]],
                    rows_v.at[pl.ds(j * 128, 128)], sem).start()

        def wait_gather(block, rows_v, sem):
            for j in range(SPB):
                pltpu.make_async_copy(
                    feat_hbm.at[idx_v.at[block * SPB + j]],
                    rows_v.at[pl.ds(j * 128, 128)], sem).wait()

        iota16 = lax.iota(jnp.int32, _L)

        def reduce_block(block, rows_v):
            @pl.loop(0, NB)
            def _row(r):
                rbase = r * S
                lo = [None] * NG
                hi = [None] * NG
                for s in range(0, S, 2):
                    for g in range(NG):
                        w0 = rows_v[rbase + s, pl.ds(g * _L, _L)]
                        w1 = rows_v[rbase + s + 1, pl.ds(g * _L, _L)]
                        pair = (plsc.bitcast(w0, jnp.bfloat16)
                                + plsc.bitcast(w1, jnp.bfloat16))
                        a, b = plsc.unpack(
                            pair, format=plsc.PackFormat.INTERLEAVED)
                        if s == 0:
                            lo[g], hi[g] = a, b
                        else:
                            lo[g], hi[g] = lo[g] + a, hi[g] + b
                orow = block * NB + r
                rsplat = jnp.full((_L,), 0, jnp.int32) + orow
                for g in range(NG):
                    cols = iota16 * 2 + (g * 2 * _L)
                    plsc.store_scatter(out_v, [rsplat, cols], lo[g] * scale)
                    plsc.store_scatter(out_v, [rsplat, cols + 1],
                                       hi[g] * scale)

        start_gather(0, rows0, sem0)

        @pl.loop(0, nblocks, step=2)
        def _blocks(i):
            start_gather(i + 1, rows1, sem1)
            wait_gather(i, rows0, sem0)
            reduce_block(i, rows0)
            start_gather(jnp.minimum(i + 2, nblocks - 1), rows0, sem0)
            wait_gather(i + 1, rows1, sem1)
            reduce_block(i + 1, rows1)

        wait_gather(0, rows0, sem0)

        valid_rows = nblocks * NB
        for c in range(n_chunks):
            @pl.when((c + 1) * chunk_rows <= valid_rows)
            def _flush():
                pltpu.sync_copy(
                    out_v.at[pl.ds(c * chunk_rows, chunk_rows)],
                    out_hbm.at[pl.ds(base_row + c * chunk_rows, chunk_rows)])

    def call(to_neighs, features):
        fb = features.astype(jnp.bfloat16).reshape(N, DW, 2)
        fpacked = lax.bitcast_convert_type(fb, jnp.int32)
        idx = to_neighs.reshape(-1).astype(jnp.int32)
        total = NW * max_blocks * NB * S
        if total > idx.size:
            idx = jnp.concatenate(
                [idx, jnp.zeros((total - idx.size,), jnp.int32)])
        return sc_mean(idx.reshape(NW, max_blocks * SPB, 128), fpacked)

    return call


def kernel(nodes, to_neighs, features, num_sample):
    B, S = to_neighs.shape
    N, D = features.shape
    return _make_sc_mean(B, S, N, D)(to_neighs, features)


# speedup vs baseline: 4.8703x; 4.8703x over previous
import functools

import jax
import jax.numpy as jnp
from jax import lax
from jax.experimental import pallas as pl
from jax.experimental.pallas import tpu as pltpu
from jax.experimental.pallas import tpu_sc as plsc


_L = 16


@functools.lru_cache(maxsize=None)
def _make_sc_mean(B: int, S: int, N: int, D: int):
    info = plsc.get_sparse_core_info()
    NC, NS = info.num_cores, info.num_subcores
    NW = NC * NS

    NB = 256 // S
    SPB = NB * S // 128
    assert 128 % S == 0 and D % _L == 0
    b_per_w = -(-B // NW)
    b_per_w = -(-b_per_w // (2 * NB)) * (2 * NB)
    max_blocks = b_per_w // NB
    tail_rows = B - (B // b_per_w) * b_per_w if B % b_per_w else b_per_w
    chunk_rows = 1
    for c in range(min(tail_rows, b_per_w), 0, -1):
        if tail_rows % c == 0 and b_per_w % c == 0:
            chunk_rows = c
            break
    n_chunks = b_per_w // chunk_rows
    bpc = chunk_rows // NB
    assert bpc * NB == chunk_rows and bpc % 2 == 0
    for w in range(NW):
        cnt = min(max_blocks, (B - w * b_per_w) // NB)
        assert cnt >= 2 and cnt % 2 == 0, (w, cnt)
    nd = D // _L
    scale = 1.0 / float(S)

    mesh = plsc.VectorSubcoreMesh(core_axis_name="c", subcore_axis_name="s")

    @functools.partial(
        pl.kernel,
        out_type=jax.ShapeDtypeStruct((B, D), jnp.float32),
        mesh=mesh,
        scratch_types=[
            pltpu.VMEM((max_blocks * SPB, 128), jnp.int32),
            pltpu.VMEM((NB * S, D), jnp.float32),
            pltpu.VMEM((NB * S, D), jnp.float32),
            pltpu.VMEM((b_per_w, D), jnp.float32),
            pltpu.SemaphoreType.DMA,
            pltpu.SemaphoreType.DMA,
            pltpu.SemaphoreType.DMA,
        ],
    )
    def sc_mean(idx_hbm, feat_hbm, out_hbm, idx_v, rows0, rows1,
                out_v, sem0, sem1, sem_out):
        wid = lax.axis_index("s") * NC + lax.axis_index("c")
        base_row = wid * b_per_w
        nblocks = jnp.minimum(max_blocks, (B - base_row) // NB)

        def start_gather(block, rows_v, sem):
            for j in range(SPB):
                pltpu.make_async_copy(
                    feat_hbm.at[idx_v.at[block * SPB + j]],
                    rows_v.at[pl.ds(j * 128, 128)], sem).start()

        def wait_gather(block, rows_v, sem):
            for j in range(SPB):
                pltpu.make_async_copy(
                    feat_hbm.at[idx_v.at[block * SPB + j]],
                    rows_v.at[pl.ds(j * 128, 128)], sem).wait()

        def reduce_block(block, rows_v):
            @pl.loop(0, NB)
            def _row(r):
                rbase = r * S
                accs = [rows_v[rbase, pl.ds(d * _L, _L)] for d in range(nd)]
                for s in range(1, S):
                    accs = [accs[d] + rows_v[rbase + s, pl.ds(d * _L, _L)]
                            for d in range(nd)]
                orow = block * NB + r
                for d in range(nd):
                    out_v[orow, pl.ds(d * _L, _L)] = accs[d] * scale

        head = 8
        pltpu.sync_copy(idx_hbm.at[wid].at[pl.ds(0, head)],
                        idx_v.at[pl.ds(0, head)])
        start_gather(0, rows0, sem0)
        pltpu.sync_copy(
            idx_hbm.at[wid].at[pl.ds(head, max_blocks * SPB - head)],
            idx_v.at[pl.ds(head, max_blocks * SPB - head)])

        def flush_chunk(c):
            off = pl.multiple_of(base_row + c * chunk_rows, 8)
            pltpu.make_async_copy(
                out_v.at[pl.ds(c * chunk_rows, chunk_rows)],
                out_hbm.at[pl.ds(off, chunk_rows)],
                sem_out).start()

        @pl.loop(0, nblocks, step=2)
        def _blocks(i):
            start_gather(i + 1, rows1, sem1)
            wait_gather(i, rows0, sem0)
            reduce_block(i, rows0)
            @pl.when(i + 2 < nblocks)
            def _prefetch():
                start_gather(i + 2, rows0, sem0)
            wait_gather(i + 1, rows1, sem1)
            reduce_block(i + 1, rows1)

            @pl.when(lax.rem(i + 2, bpc) == 0)
            def _maybe_flush():
                flush_chunk((i + 2) // bpc - 1)

        @pl.loop(0, nblocks // bpc)
        def _drain(c):
            pltpu.make_async_copy(
                out_v.at[pl.ds(0, chunk_rows)],
                out_hbm.at[pl.ds(base_row, chunk_rows)], sem_out).wait()

    def call(to_neighs, features):
        idx = to_neighs.reshape(-1).astype(jnp.int32)
        total = NW * max_blocks * NB * S
        if total > idx.size:
            idx = jnp.concatenate(
                [idx, jnp.zeros((total - idx.size,), jnp.int32)])
        return sc_mean(idx.reshape(NW, max_blocks * SPB, 128), features)

    return call


def kernel(nodes, to_neighs, features, num_sample):
    B, S = to_neighs.shape
    N, D = features.shape
    return _make_sc_mean(B, S, N, D)(to_neighs, features)


# speedup vs baseline: 4.8727x; 1.0005x over previous
import functools

import jax
import jax.numpy as jnp
from jax import lax
from jax.experimental import pallas as pl
from jax.experimental.pallas import tpu as pltpu
from jax.experimental.pallas import tpu_sc as plsc


_L = 16


@functools.lru_cache(maxsize=None)
def _make_sc_mean(B: int, S: int, N: int, D: int):
    info = plsc.get_sparse_core_info()
    NC, NS = info.num_cores, info.num_subcores
    NW = NC * NS

    NB = 256 // S
    SPB = NB * S // 128
    assert 128 % S == 0 and D % _L == 0
    b_per_w = -(-B // NW)
    b_per_w = -(-b_per_w // (2 * NB)) * (2 * NB)
    max_blocks = b_per_w // NB
    tail_rows = B - (B // b_per_w) * b_per_w if B % b_per_w else b_per_w
    chunk_rows = 1
    for c in range(min(tail_rows, b_per_w), 0, -1):
        if tail_rows % c == 0 and b_per_w % c == 0:
            chunk_rows = c
            break
    n_chunks = b_per_w // chunk_rows
    bpc = chunk_rows // NB
    assert bpc * NB == chunk_rows and bpc % 2 == 0
    for w in range(NW):
        cnt = min(max_blocks, (B - w * b_per_w) // NB)
        assert cnt >= 2 and cnt % 2 == 0, (w, cnt)
    nd = D // _L
    scale = 1.0 / float(S)

    mesh = plsc.VectorSubcoreMesh(core_axis_name="c", subcore_axis_name="s")

    @functools.partial(
        pl.kernel,
        out_type=jax.ShapeDtypeStruct((B, D), jnp.float32),
        mesh=mesh,
        compiler_params=pltpu.CompilerParams(skip_device_barrier=True),
        scratch_types=[
            pltpu.VMEM((max_blocks * SPB, 128), jnp.int32),
            pltpu.VMEM((NB * S, D), jnp.float32),
            pltpu.VMEM((NB * S, D), jnp.float32),
            pltpu.VMEM((b_per_w, D), jnp.float32),
            pltpu.SemaphoreType.DMA,
            pltpu.SemaphoreType.DMA,
            pltpu.SemaphoreType.DMA,
        ],
    )
    def sc_mean(idx_hbm, feat_hbm, out_hbm, idx_v, rows0, rows1,
                out_v, sem0, sem1, sem_out):
        wid = lax.axis_index("s") * NC + lax.axis_index("c")
        base_row = wid * b_per_w
        nblocks = jnp.minimum(max_blocks, (B - base_row) // NB)

        def start_gather(block, rows_v, sem):
            for j in range(SPB):
                pltpu.make_async_copy(
                    feat_hbm.at[idx_v.at[block * SPB + j]],
                    rows_v.at[pl.ds(j * 128, 128)], sem).start()

        def wait_gather(block, rows_v, sem):
            for j in range(SPB):
                pltpu.make_async_copy(
                    feat_hbm.at[idx_v.at[block * SPB + j]],
                    rows_v.at[pl.ds(j * 128, 128)], sem).wait()

        def reduce_block(block, rows_v):
            @pl.loop(0, NB)
            def _row(r):
                rbase = r * S
                accs = [rows_v[rbase, pl.ds(d * _L, _L)] for d in range(nd)]
                for s in range(1, S):
                    accs = [accs[d] + rows_v[rbase + s, pl.ds(d * _L, _L)]
                            for d in range(nd)]
                orow = block * NB + r
                for d in range(nd):
                    out_v[orow, pl.ds(d * _L, _L)] = accs[d] * scale

        head = 8
        pltpu.sync_copy(idx_hbm.at[wid].at[pl.ds(0, head)],
                        idx_v.at[pl.ds(0, head)])
        start_gather(0, rows0, sem0)
        pltpu.sync_copy(
            idx_hbm.at[wid].at[pl.ds(head, max_blocks * SPB - head)],
            idx_v.at[pl.ds(head, max_blocks * SPB - head)])

        def flush_chunk(c):
            off = pl.multiple_of(base_row + c * chunk_rows, 8)
            pltpu.make_async_copy(
                out_v.at[pl.ds(c * chunk_rows, chunk_rows)],
                out_hbm.at[pl.ds(off, chunk_rows)],
                sem_out).start()

        @pl.loop(0, nblocks, step=2)
        def _blocks(i):
            start_gather(i + 1, rows1, sem1)
            wait_gather(i, rows0, sem0)
            reduce_block(i, rows0)
            @pl.when(i + 2 < nblocks)
            def _prefetch():
                start_gather(i + 2, rows0, sem0)
            wait_gather(i + 1, rows1, sem1)
            reduce_block(i + 1, rows1)

            @pl.when(lax.rem(i + 2, bpc) == 0)
            def _maybe_flush():
                flush_chunk((i + 2) // bpc - 1)

        @pl.loop(0, nblocks // bpc)
        def _drain(c):
            pltpu.make_async_copy(
                out_v.at[pl.ds(0, chunk_rows)],
                out_hbm.at[pl.ds(base_row, chunk_rows)], sem_out).wait()

    def call(to_neighs, features):
        idx = to_neighs.reshape(-1).astype(jnp.int32)
        total = NW * max_blocks * NB * S
        if total > idx.size:
            idx = jnp.concatenate(
                [idx, jnp.zeros((total - idx.size,), jnp.int32)])
        return sc_mean(idx.reshape(NW, max_blocks * SPB, 128), features)

    return call


def kernel(nodes, to_neighs, features, num_sample):
    B, S = to_neighs.shape
    N, D = features.shape
    return _make_sc_mean(B, S, N, D)(to_neighs, features)


# speedup vs baseline: 4.9817x; 1.0224x over previous
import functools

import jax
import jax.numpy as jnp
from jax import lax
from jax.experimental import pallas as pl
from jax.experimental.pallas import tpu as pltpu
from jax.experimental.pallas import tpu_sc as plsc


_L = 16


@functools.lru_cache(maxsize=None)
def _make_sc_mean(B: int, S: int, N: int, D: int):
    info = plsc.get_sparse_core_info()
    NC, NS = info.num_cores, info.num_subcores
    NW = NC * NS

    NB = 256 // S
    SPB = NB * S // 128
    assert 128 % S == 0 and D % _L == 0
    b_per_w = -(-B // NW)
    b_per_w = -(-b_per_w // (2 * NB)) * (2 * NB)
    max_blocks = b_per_w // NB
    tail_rows = B - (B // b_per_w) * b_per_w if B % b_per_w else b_per_w
    chunk_rows = 1
    for c in range(min(tail_rows, b_per_w), 0, -1):
        if tail_rows % c == 0 and b_per_w % c == 0:
            chunk_rows = c
            break
    n_chunks = b_per_w // chunk_rows
    bpc = chunk_rows // NB
    assert bpc * NB == chunk_rows and bpc % 2 == 0
    tail_blocks = tail_rows // NB
    assert (NW - 1) * b_per_w + tail_rows <= B
    assert tail_blocks >= 1 and (NB * S) % 8 == 0
    for w in range(NW):
        cnt = min(max_blocks, (B - w * b_per_w) // NB)
        assert cnt >= 2 and cnt % 2 == 0, (w, cnt)
    nd = D // _L
    scale = 1.0 / float(S)

    mesh = plsc.VectorSubcoreMesh(core_axis_name="c", subcore_axis_name="s")

    @functools.partial(
        pl.kernel,
        out_type=jax.ShapeDtypeStruct((B, D), jnp.float32),
        mesh=mesh,
        scratch_types=[
            pltpu.VMEM((b_per_w * S,), jnp.int32),
            pltpu.VMEM((NB * S, D), jnp.float32),
            pltpu.VMEM((NB * S, D), jnp.float32),
            pltpu.VMEM((b_per_w, D), jnp.float32),
            pltpu.SemaphoreType.DMA,
            pltpu.SemaphoreType.DMA,
            pltpu.SemaphoreType.DMA,
        ],
    )
    def sc_mean(idx_hbm, feat_hbm, out_hbm, idx_v, rows0, rows1,
                out_v, sem0, sem1, sem_out):
        wid = lax.axis_index("s") * NC + lax.axis_index("c")
        base_row = wid * b_per_w
        nblocks = jnp.minimum(max_blocks, (B - base_row) // NB)

        def start_gather(block, rows_v, sem):
            for j in range(SPB):
                off = pl.multiple_of(block * (NB * S) + j * 128, 128)
                pltpu.make_async_copy(
                    feat_hbm.at[idx_v.at[pl.ds(off, 128)]],
                    rows_v.at[pl.ds(j * 128, 128)], sem).start()

        def wait_gather(block, rows_v, sem):
            for j in range(SPB):
                off = pl.multiple_of(block * (NB * S) + j * 128, 128)
                pltpu.make_async_copy(
                    feat_hbm.at[idx_v.at[pl.ds(off, 128)]],
                    rows_v.at[pl.ds(j * 128, 128)], sem).wait()

        def reduce_block(block, rows_v):
            @pl.loop(0, NB)
            def _row(r):
                rbase = r * S
                accs = [rows_v[rbase, pl.ds(d * _L, _L)] for d in range(nd)]
                for s in range(1, S):
                    accs = [accs[d] + rows_v[rbase + s, pl.ds(d * _L, _L)]
                            for d in range(nd)]
                orow = block * NB + r
                for d in range(nd):
                    out_v[orow, pl.ds(d * _L, _L)] = accs[d] * scale

        ibase = base_row * S
        head = NB * S
        pltpu.sync_copy(idx_hbm.at[pl.ds(ibase, head)],
                        idx_v.at[pl.ds(0, head)])
        start_gather(0, rows0, sem0)

        @pl.when(nblocks == max_blocks)
        def _stage_full():
            pltpu.sync_copy(
                idx_hbm.at[pl.ds(ibase + head, max_blocks * NB * S - head)],
                idx_v.at[pl.ds(head, max_blocks * NB * S - head)])

        @pl.when(nblocks != max_blocks)
        def _stage_tail():
            pltpu.sync_copy(
                idx_hbm.at[pl.ds(ibase + head, tail_blocks * NB * S - head)],
                idx_v.at[pl.ds(head, tail_blocks * NB * S - head)])

        def flush_chunk(c):
            off = pl.multiple_of(base_row + c * chunk_rows, 8)
            pltpu.make_async_copy(
                out_v.at[pl.ds(c * chunk_rows, chunk_rows)],
                out_hbm.at[pl.ds(off, chunk_rows)],
                sem_out).start()

        @pl.loop(0, nblocks, step=2)
        def _blocks(i):
            start_gather(i + 1, rows1, sem1)
            wait_gather(i, rows0, sem0)
            reduce_block(i, rows0)
            @pl.when(i + 2 < nblocks)
            def _prefetch():
                start_gather(i + 2, rows0, sem0)
            wait_gather(i + 1, rows1, sem1)
            reduce_block(i + 1, rows1)

            @pl.when(lax.rem(i + 2, bpc) == 0)
            def _maybe_flush():
                flush_chunk((i + 2) // bpc - 1)

        @pl.loop(0, nblocks // bpc)
        def _drain(c):
            pltpu.make_async_copy(
                out_v.at[pl.ds(0, chunk_rows)],
                out_hbm.at[pl.ds(base_row, chunk_rows)], sem_out).wait()

    def call(to_neighs, features):
        return sc_mean(to_neighs.reshape(-1).astype(jnp.int32), features)

    return call


def kernel(nodes, to_neighs, features, num_sample):
    B, S = to_neighs.shape
    N, D = features.shape
    return _make_sc_mean(B, S, N, D)(to_neighs, features)
